# baseline (speedup 1.0000x reference)
import jax
import jax.numpy as jnp
from jax import lax
from jax.experimental import pallas as pl
from jax.experimental.pallas import tpu as pltpu
from jax.experimental.pallas import tpu_sc as plsc

_NC = 2
_NS = 16
_NW = _NC * _NS

_ROWS = 384
_HW = 50176
_HALF = _HW // 2
_RPT = _ROWS // _NW
_NCH = _RPT * 2
_NSLOT = 4

_RB = 32


def _sc_body(x_hbm, out_hbm, b0, b1, b2, b3, i0, i1, i2, i3, o0, o1, o2, o3):
    wid = lax.axis_index("s") * _NC + lax.axis_index("c")
    row0 = wid * _RPT

    bufs = (b0, b1, b2, b3)
    isem = (i0, i1, i2, i3)
    osem = (o0, o1, o2, o3)

    def src(c):
        r = row0 + (c // 2)
        return x_hbm.at[r, pl.ds((c % 2) * _HALF, _HALF)]

    def dst(c):
        r = row0 + (c // 2)
        return out_hbm.at[r, pl.ds((c % 2) * _HALF, _HALF)]

    for s in range(_NSLOT):
        pltpu.async_copy(src(s), bufs[s], isem[s])

    for c in range(_NCH):
        s = c % _NSLOT
        pltpu.make_async_copy(src(c), bufs[s], isem[s]).wait()

        buf = bufs[s]

        @plsc.parallel_loop(0, _HALF, step=16, unroll=8)
        def _(i):
            buf[pl.ds(i, 16)] = buf[pl.ds(i, 16)] * 10.0

        pltpu.async_copy(bufs[s], dst(c), osem[s])
        if c + _NSLOT < _NCH:
            pltpu.make_async_copy(bufs[s], dst(c), osem[s]).wait()
            pltpu.async_copy(src(c + _NSLOT), bufs[s], isem[s])

    for c in range(_NCH - _NSLOT, _NCH):
        s = c % _NSLOT
        pltpu.make_async_copy(bufs[s], dst(c), osem[s]).wait()


def _tc_body(x_ref, out_ref):
    out_ref[...] = x_ref[...] * 10.0


def kernel(x, sal_map):
    B, C, H, W = x.shape
    xr = x.reshape(2, _ROWS, _HW)

    tc_half = pl.pallas_call(
        _tc_body,
        grid=(_ROWS // _RB,),
        in_specs=[pl.BlockSpec((_RB, _HW), lambda i: (i, 0))],
        out_specs=pl.BlockSpec((_RB, _HW), lambda i: (i, 0)),
        out_shape=jax.ShapeDtypeStruct((_ROWS, _HW), jnp.float32),
    )(xr[0])

    mesh = plsc.VectorSubcoreMesh(
        core_axis_name="c", subcore_axis_name="s",
        num_cores=_NC, num_subcores=_NS)

    sc_half = pl.kernel(
        _sc_body,
        mesh=mesh,
        out_type=jax.ShapeDtypeStruct((_ROWS, _HW), jnp.float32),
        scratch_types=(
            [pltpu.VMEM((_HALF,), jnp.float32)] * _NSLOT
            + [pltpu.SemaphoreType.DMA] * (2 * _NSLOT)
        ),
    )(xr[1])

    xm = jnp.stack([tc_half, sc_half], axis=0)
    return xm.reshape(B, C, H, W), sal_map


# speedup vs baseline: 1.4073x; 1.4073x over previous
import functools

import jax
import jax.numpy as jnp
from jax import lax
from jax.experimental import pallas as pl
from jax.experimental.pallas import tpu as pltpu
from jax.experimental.pallas import tpu_sc as plsc

KEEP_PERCENT = 0.1
SCALE = 1.0 / KEEP_PERCENT
DROP_PERCENT = 1.0 - KEEP_PERCENT

_NC = 2
_NS = 16
_NW = _NC * _NS
_NSLOT = 4


def _monotone_key(f):
    v = jax.lax.bitcast_convert_type(f, jnp.int32)
    return v ^ ((v >> 31) & jnp.int32(0x7FFFFFFF))


def _key_to_float(k):
    v = k ^ ((k >> 31) & jnp.int32(0x7FFFFFFF))
    return jax.lax.bitcast_convert_type(v, jnp.float32)


def _thresh_body(rank, sm_ref, thr_ref):
    sm = sm_ref[...]
    B = sm.shape[0]
    keys = _monotone_key(sm)
    target = jnp.int32(rank + 1)

    def count_lt(mid):
        return jnp.sum((keys < mid).astype(jnp.int32), axis=(1, 2),
                       keepdims=True)

    c = count_lt(jnp.int32(0))
    p0 = jnp.where(c >= target, jnp.int32(-2147483648), jnp.int32(0))

    def step(i, p):
        bit = 30 - i
        mid = p + (jnp.int32(1) << bit)
        c = count_lt(mid)
        return jnp.where(c >= target, p, mid)

    p = jax.lax.fori_loop(0, 31, step, p0)
    thr = _key_to_float(p)
    thr_ref[...] = jnp.broadcast_to(thr, (B, 1, 128))


def _sc_body(rows, hw, rpt, x_hbm, sal_hbm, thr_hbm, out_hbm, drop_hbm,
             salbuf, b0, b1, b2, b3, thrbuf,
             i0, i1, i2, i3, o0, o1, o2, o3, dsem, tsem):
    half = hw // 2
    tiles_per_b = 96 // rpt

    wid = lax.axis_index("s") * _NC + lax.axis_index("c")
    b = wid // tiles_per_b
    cslot = wid % tiles_per_b
    row0 = b * 96 + cslot * rpt

    bufs = (b0, b1, b2, b3)
    isem = (i0, i1, i2, i3)
    osem = (o0, o1, o2, o3)

    pltpu.async_copy(thr_hbm.at[b, pl.ds(0, 16)], thrbuf, tsem).wait()
    tv = thrbuf[...]

    for h in range(2):
        def src(c, _h=h):
            return x_hbm.at[row0 + c, pl.ds(_h * half, half)]

        def dst(c, _h=h):
            return out_hbm.at[row0 + c, pl.ds(_h * half, half)]

        pltpu.async_copy(sal_hbm.at[b, pl.ds(h * half, half)], salbuf,
                         tsem).wait()

        if True:
            @pl.when(cslot == 0)
            def _emit_drop():
                @plsc.parallel_loop(0, half, step=16, unroll=8)
                def _(i):
                    sv = salbuf[pl.ds(i, 16)]
                    salbuf[pl.ds(i, 16)] = jnp.where(
                        sv > tv, jnp.float32(1.0), jnp.float32(0.0))

                pltpu.async_copy(
                    salbuf, drop_hbm.at[b, pl.ds(h * half, half)],
                    dsem).wait()

                @plsc.parallel_loop(0, half, step=16, unroll=8)
                def _(i):
                    salbuf[pl.ds(i, 16)] = salbuf[pl.ds(i, 16)] * SCALE

            @pl.when(cslot != 0)
            def _build_mask():
                @plsc.parallel_loop(0, half, step=16, unroll=8)
                def _(i):
                    sv = salbuf[pl.ds(i, 16)]
                    salbuf[pl.ds(i, 16)] = jnp.where(
                        sv > tv, jnp.float32(SCALE), jnp.float32(0.0))

        for s in range(_NSLOT):
            pltpu.async_copy(src(s), bufs[s], isem[s])

        for c in range(rpt):
            s = c % _NSLOT
            pltpu.make_async_copy(src(c), bufs[s], isem[s]).wait()

            buf = bufs[s]

            @plsc.parallel_loop(0, half, step=16, unroll=8)
            def _(i):
                buf[pl.ds(i, 16)] = buf[pl.ds(i, 16)] * salbuf[pl.ds(i, 16)]

            pltpu.async_copy(bufs[s], dst(c), osem[s])
            if c + _NSLOT < rpt:
                pltpu.make_async_copy(bufs[s], dst(c), osem[s]).wait()
                pltpu.async_copy(src(c + _NSLOT), bufs[s], isem[s])

        for c in range(rpt - _NSLOT, rpt):
            s = c % _NSLOT
            pltpu.make_async_copy(bufs[s], dst(c), osem[s]).wait()


def kernel(x, sal_map):
    B, C, H, W = x.shape
    hw = H * W
    rank = int(hw * DROP_PERCENT)
    s0 = 8
    s1 = hw // s0
    rows = B * C
    rpt = rows // _NW
    half = hw // 2

    xr = x.reshape(rows, hw)
    sm3 = sal_map.reshape(B, s0, s1)
    sm2 = sal_map.reshape(B, hw)

    thr = pl.pallas_call(
        functools.partial(_thresh_body, rank),
        out_shape=jax.ShapeDtypeStruct((B, 1, 128), jnp.float32),
    )(sm3)

    mesh = plsc.VectorSubcoreMesh(
        core_axis_name="c", subcore_axis_name="s",
        num_cores=_NC, num_subcores=_NS)

    k = pl.kernel(
        functools.partial(_sc_body, rows, hw, rpt),
        mesh=mesh,
        out_type=[
            jax.ShapeDtypeStruct((rows, hw), jnp.float32),
            jax.ShapeDtypeStruct((B, hw), jnp.float32),
        ],
        scratch_types=(
            [pltpu.VMEM((half,), jnp.float32)]
            + [pltpu.VMEM((half,), jnp.float32)] * _NSLOT
            + [pltpu.VMEM((16,), jnp.float32)]
            + [pltpu.SemaphoreType.DMA] * (2 * _NSLOT + 2)
        ),
    )

    xm, drop = k(xr, sm2, thr.reshape(B, 128))
    return xm.reshape(B, C, H, W), drop.reshape(B, H, W)


# speedup vs baseline: 1.4243x; 1.0121x over previous
import functools

import jax
import jax.numpy as jnp
from jax import lax
from jax.experimental import pallas as pl
from jax.experimental.pallas import tpu as pltpu
from jax.experimental.pallas import tpu_sc as plsc

KEEP_PERCENT = 0.1
SCALE = 1.0 / KEEP_PERCENT
DROP_PERCENT = 1.0 - KEEP_PERCENT

_NC = 2
_NS = 16
_NW = _NC * _NS
_NSLOT = 4


def _monotone_key(f):
    v = jax.lax.bitcast_convert_type(f, jnp.int32)
    return v ^ ((v >> 31) & jnp.int32(0x7FFFFFFF))


def _key_to_float(k):
    v = k ^ ((k >> 31) & jnp.int32(0x7FFFFFFF))
    return jax.lax.bitcast_convert_type(v, jnp.float32)


def _thresh_body(rank, sm_ref, thr_ref):
    sm = sm_ref[...]
    B = sm.shape[0]
    keys = _monotone_key(sm)
    target = jnp.int32(rank + 1)

    def count_lt(mid):
        return jnp.sum((keys < mid).astype(jnp.int32), axis=(1, 2),
                       keepdims=True)

    c = count_lt(jnp.int32(0))
    p0 = jnp.where(c >= target, jnp.int32(-2147483648), jnp.int32(0))

    def step(i, p):
        bit = 30 - i
        mid = p + (jnp.int32(1) << bit)
        c = count_lt(mid)
        return jnp.where(c >= target, p, mid)

    p = jax.lax.fori_loop(0, 31, step, p0)
    thr = _key_to_float(p)
    thr_ref[...] = jnp.broadcast_to(thr, (B, 1, 128))


def _sc_body(rows, hw, rpt, x_hbm, sal_hbm, thr_hbm, out_hbm, drop_hbm,
             salbuf, b0, b1, b2, b3, thrbuf,
             i0, i1, i2, i3, o0, o1, o2, o3, dsem, tsem, ssem):
    half = hw // 2
    nch = 2 * rpt
    tiles_per_b = 96 // rpt

    wid = lax.axis_index("s") * _NC + lax.axis_index("c")
    b = wid // tiles_per_b
    cslot = wid % tiles_per_b
    row0 = b * 96 + cslot * rpt

    bufs = (b0, b1, b2, b3)
    isem = (i0, i1, i2, i3)
    osem = (o0, o1, o2, o3)

    def src(c):
        return x_hbm.at[row0 + (c % rpt), pl.ds((c // rpt) * half, half)]

    def dst(c):
        return out_hbm.at[row0 + (c % rpt), pl.ds((c // rpt) * half, half)]

    def sal_copy(h):
        return pltpu.async_copy(sal_hbm.at[b, pl.ds(h * half, half)],
                                salbuf, ssem)

    pltpu.async_copy(thr_hbm.at[b, pl.ds(0, 16)], thrbuf, tsem)
    sal_copy(0)
    for s in range(_NSLOT):
        pltpu.async_copy(src(s), bufs[s], isem[s])

    pltpu.make_async_copy(thr_hbm.at[b, pl.ds(0, 16)], thrbuf, tsem).wait()
    tv = thrbuf[...]

    def build_mask(h):
        pltpu.make_async_copy(sal_hbm.at[b, pl.ds(h * half, half)],
                              salbuf, ssem).wait()

        @pl.when(cslot == 0)
        def _emit_drop():
            @plsc.parallel_loop(0, half, step=16, unroll=8)
            def _(i):
                sv = salbuf[pl.ds(i, 16)]
                salbuf[pl.ds(i, 16)] = jnp.where(
                    sv > tv, jnp.float32(1.0), jnp.float32(0.0))

            pltpu.async_copy(
                salbuf, drop_hbm.at[b, pl.ds(h * half, half)], dsem).wait()

            @plsc.parallel_loop(0, half, step=16, unroll=8)
            def _(i):
                salbuf[pl.ds(i, 16)] = salbuf[pl.ds(i, 16)] * SCALE

        @pl.when(cslot != 0)
        def _scaled_mask():
            @plsc.parallel_loop(0, half, step=16, unroll=8)
            def _(i):
                sv = salbuf[pl.ds(i, 16)]
                salbuf[pl.ds(i, 16)] = jnp.where(
                    sv > tv, jnp.float32(SCALE), jnp.float32(0.0))

    build_mask(0)

    for c in range(nch):
        s = c % _NSLOT
        if c == rpt:
            build_mask(1)
        pltpu.make_async_copy(src(c), bufs[s], isem[s]).wait()

        buf = bufs[s]

        @plsc.parallel_loop(0, half, step=16, unroll=8)
        def _(i):
            buf[pl.ds(i, 16)] = buf[pl.ds(i, 16)] * salbuf[pl.ds(i, 16)]

        if c == rpt - 1:
            sal_copy(1)
        pltpu.async_copy(bufs[s], dst(c), osem[s])
        if c + _NSLOT < nch:
            pltpu.make_async_copy(bufs[s], dst(c), osem[s]).wait()
            pltpu.async_copy(src(c + _NSLOT), bufs[s], isem[s])

    for c in range(nch - _NSLOT, nch):
        s = c % _NSLOT
        pltpu.make_async_copy(bufs[s], dst(c), osem[s]).wait()


def kernel(x, sal_map):
    B, C, H, W = x.shape
    hw = H * W
    rank = int(hw * DROP_PERCENT)
    s0 = 8
    s1 = hw // s0
    rows = B * C
    rpt = rows // _NW
    half = hw // 2

    xr = x.reshape(rows, hw)
    sm3 = sal_map.reshape(B, s0, s1)
    sm2 = sal_map.reshape(B, hw)

    thr = pl.pallas_call(
        functools.partial(_thresh_body, rank),
        out_shape=jax.ShapeDtypeStruct((B, 1, 128), jnp.float32),
    )(sm3)

    mesh = plsc.VectorSubcoreMesh(
        core_axis_name="c", subcore_axis_name="s",
        num_cores=_NC, num_subcores=_NS)

    k = pl.kernel(
        functools.partial(_sc_body, rows, hw, rpt),
        mesh=mesh,
        out_type=[
            jax.ShapeDtypeStruct((rows, hw), jnp.float32),
            jax.ShapeDtypeStruct((B, hw), jnp.float32),
        ],
        scratch_types=(
            [pltpu.VMEM((half,), jnp.float32)]
            + [pltpu.VMEM((half,), jnp.float32)] * _NSLOT
            + [pltpu.VMEM((16,), jnp.float32)]
            + [pltpu.SemaphoreType.DMA] * (2 * _NSLOT + 3)
        ),
    )

    xm, drop = k(xr, sm2, thr.reshape(B, 128))
    return xm.reshape(B, C, H, W), drop.reshape(B, H, W)
